# Initial kernel scaffold; baseline (speedup 1.0000x reference)
import jax
import jax.numpy as jnp
from jax.experimental import pallas as pl


def kernel(key, is_training, c_rec, f_rec, oc_rec, of_rec, e_rec, s_rec, r_rec, m_rec, c_lig, f_lig, oc_lig, of_lig, e_lig, s_lig, r_lig, m_lig, W_e1, b_e1, g_eln1, be_eln1, W_e2, b_e2, W_Q, W_K, W_V, g_nln1, b_nln1, W_n1, b_n1, g_nln2, b_nln2, W_n2, b_n2, g_nln3, b_nln3, W_c1, b_c1, g_cln1, b_cln1, W_c2, b_c2):
    raise NotImplementedError("write your pallas kernel here")


# speedup vs baseline: 8.8026x; 8.8026x over previous
import functools

import jax
import jax.numpy as jnp
from jax import lax
from jax.experimental import pallas as pl
from jax.experimental.pallas import tpu as pltpu

B, N, E, D, EDI = 8, 512, 5120, 128, 27
SLOPE = 0.01
C_W = 0.3
F_W = 0.3
EB = 512
NEB = E // EB
F32 = jnp.float32


def _lrelu(x):
    return jnp.where(x >= 0, x, SLOPE * x)


def _ln(x, g, b, eps=1e-5):
    mu = jnp.mean(x, axis=-1, keepdims=True)
    var = jnp.mean((x - mu) * (x - mu), axis=-1, keepdims=True)
    return (x - mu) * jax.lax.rsqrt(var + eps) * g + b


def _dot(a, b):
    return jnp.dot(a, b, preferred_element_type=F32)


def _edge_body(r_col, s_col, r_row, f, c8, e32, w1fr, w1fs, w1d, w1e, invsig,
               b_e1, g_eln1, be_eln1, w_e2, b_e2, w_c1, b_c1, g_cln1, b_cln1,
               w_c2, b_c2p, out_msum, out_aux, g1, g2):
    eb = pl.program_id(1)

    @pl.when(eb == 0)
    def _():
        g1[...] = _dot(f[0], w1fr[...])
        g2[...] = _dot(f[0], w1fs[...])

    idx_r = r_col[0, 0, :, :]
    idx_s = s_col[0, 0, :, :]
    idx_rr = r_row[0, 0, :, :]

    iota_n = lax.broadcasted_iota(jnp.int32, (EB, N), 1)
    oh_r = (idx_r == iota_n).astype(F32)
    oh_s = (idx_s == iota_n).astype(F32)
    iota_c = lax.broadcasted_iota(jnp.int32, (N, EB), 0)
    oh_n = (iota_c == idx_rr).astype(F32)

    g1r = _dot(oh_r, g1[...])
    g2s = _dot(oh_s, g2[...])
    cr = _dot(oh_r, c8[0])
    cs = _dot(oh_s, c8[0])
    rel = cr - cs
    d2 = jnp.sum(rel * rel, axis=-1, keepdims=True)
    dist = jnp.exp(-d2 * invsig[...])

    x = g1r + g2s + _dot(dist, w1d[...]) + _dot(e32[0], w1e[...]) + b_e1[...]
    x = _ln(_lrelu(x), g_eln1[...], be_eln1[...])
    msg = _dot(x, w_e2[...]) + b_e2[...]

    cw = _ln(_lrelu(_dot(msg, w_c1[...]) + b_c1[...]), g_cln1[...], b_cln1[...])
    coef = (_dot(cw, w_c2[...]) + b_c2p[...])[:, 0:1]
    cnt_lane = (lax.broadcasted_iota(jnp.int32, (EB, 8), 1) == 3).astype(F32)
    aux = coef * rel + cnt_lane

    msum = _dot(oh_n, msg)
    asum = _dot(oh_n, aux)

    @pl.when(eb == 0)
    def _():
        out_msum[0] = msum
        out_aux[0] = asum

    @pl.when(eb != 0)
    def _():
        out_msum[0] += msum
        out_aux[0] += asum


def _att_body(fq, fk, mq, mkT, wq, wk, wv, out):
    q = _lrelu(_dot(fq[0], wq[...]))
    k = _lrelu(_dot(fk[0], wk[...]))
    v = _dot(fk[0], wv[...])
    logits = lax.dot_general(q, k, (((1,), (1,)), ((), ())),
                             preferred_element_type=F32)
    mask = mq[0] * mkT[0]
    a = mask * logits - 1000.0 * (1.0 - mask)
    a = a - jnp.max(a, axis=-1, keepdims=True)
    ea = jnp.exp(a)
    a = ea / jnp.sum(ea, axis=-1, keepdims=True)
    out[0] = _dot(a, v)


def _node_body(c8, f, of, m, cross, msum, aux, g_nln1, b_nln1, wn1_f, wn1_agg,
               wn1_cross, wn1_of, b_n1, g_nln2, b_nln2, w_n2, b_n2, g_nln3,
               b_nln3, out_c, out_f):
    cnt = aux[0][:, 3:4]
    denom = jnp.maximum(cnt, 1.0)
    agg = _ln(msum[0] / denom, g_nln1[...], b_nln1[...])
    trans = aux[0] / denom
    out_c[0] = (c8[0] + C_W * trans) * m[0]

    h = _lrelu(_dot(f[0], wn1_f[...]) + _dot(agg, wn1_agg[...]) +
               _dot(cross[0], wn1_cross[...]) + _dot(of[0], wn1_of[...]) +
               b_n1[...])
    h = _ln(h, g_nln2[...], b_nln2[...])
    h = _ln(_dot(h, w_n2[...]) + b_n2[...], g_nln3[...], b_nln3[...])
    out_f[0] = (F_W * h + (1.0 - F_W) * f[0]) * m[0]


def _full(i):
    return pl.BlockSpec(i.shape, lambda *_: (0,) * len(i.shape))


def kernel(key, is_training, c_rec, f_rec, oc_rec, of_rec, e_rec, s_rec,
           r_rec, m_rec, c_lig, f_lig, oc_lig, of_lig, e_lig, s_lig, r_lig,
           m_lig, W_e1, b_e1, g_eln1, be_eln1, W_e2, b_e2, W_Q, W_K, W_V,
           g_nln1, b_nln1, W_n1, b_n1, g_nln2, b_nln2, W_n2, b_n2, g_nln3,
           b_nln3, W_c1, b_c1, g_cln1, b_cln1, W_c2, b_c2):
    TB = 2 * B
    f_all = jnp.concatenate([f_rec, f_lig], axis=0)
    of_all = jnp.concatenate([of_rec, of_lig], axis=0)
    c_all = jnp.concatenate([c_rec, c_lig], axis=0)
    c8 = jnp.pad(c_all, ((0, 0), (0, 0), (0, 5)))
    e_all = jnp.concatenate([e_rec, e_lig], axis=0)
    e32 = jnp.pad(e_all, ((0, 0), (0, 0), (0, 32 - EDI)))
    r_all = jnp.concatenate([r_rec, r_lig], axis=0)
    s_all = jnp.concatenate([s_rec, s_lig], axis=0)
    m_all = jnp.concatenate([m_rec, m_lig], axis=0)[..., None]

    r_col = r_all.reshape(TB, NEB, EB, 1)
    s_col = s_all.reshape(TB, NEB, EB, 1)
    r_row = r_all.reshape(TB, NEB, 1, EB)

    w1fr = W_e1[0:D]
    w1fs = W_e1[D:2 * D]
    w1d = jnp.pad(W_e1[2 * D:2 * D + 15], ((0, 1), (0, 0)))
    w1e = jnp.pad(W_e1[2 * D + 15:], ((0, 32 - EDI), (0, 0)))
    invsig = jnp.pad((1.0 / 1.5) ** jnp.arange(15, dtype=F32),
                     (0, 1)).reshape(1, 16)
    w_c2p = jnp.pad(W_c2, ((0, 0), (0, 7)))
    b_c2p = jnp.pad(b_c2, (0, 7)).reshape(1, 8)

    def row(v):
        return v.reshape(1, -1)

    edge_weights = [w1fr, w1fs, w1d, w1e, invsig, row(b_e1), row(g_eln1),
                    row(be_eln1), W_e2, row(b_e2), W_c1, row(b_c1),
                    row(g_cln1), row(b_cln1), w_c2p, b_c2p]
    edge = pl.pallas_call(
        _edge_body,
        grid=(TB, NEB),
        in_specs=[
            pl.BlockSpec((1, 1, EB, 1), lambda sb, eb: (sb, eb, 0, 0)),
            pl.BlockSpec((1, 1, EB, 1), lambda sb, eb: (sb, eb, 0, 0)),
            pl.BlockSpec((1, 1, 1, EB), lambda sb, eb: (sb, eb, 0, 0)),
            pl.BlockSpec((1, N, D), lambda sb, eb: (sb, 0, 0)),
            pl.BlockSpec((1, N, 8), lambda sb, eb: (sb, 0, 0)),
            pl.BlockSpec((1, EB, 32), lambda sb, eb: (sb, eb, 0)),
        ] + [_full(w) for w in edge_weights],
        out_specs=[
            pl.BlockSpec((1, N, D), lambda sb, eb: (sb, 0, 0)),
            pl.BlockSpec((1, N, 8), lambda sb, eb: (sb, 0, 0)),
        ],
        out_shape=[
            jax.ShapeDtypeStruct((TB, N, D), F32),
            jax.ShapeDtypeStruct((TB, N, 8), F32),
        ],
        scratch_shapes=[pltpu.VMEM((N, D), F32), pltpu.VMEM((N, D), F32)],
    )
    msum, aux = edge(r_col, s_col, r_row, f_all, c8, e32, *edge_weights)

    fk_all = jnp.concatenate([f_lig, f_rec], axis=0)
    mk_T = jnp.concatenate([m_lig, m_rec], axis=0)[:, None, :]
    att = pl.pallas_call(
        _att_body,
        grid=(TB,),
        in_specs=[
            pl.BlockSpec((1, N, D), lambda sb: (sb, 0, 0)),
            pl.BlockSpec((1, N, D), lambda sb: (sb, 0, 0)),
            pl.BlockSpec((1, N, 1), lambda sb: (sb, 0, 0)),
            pl.BlockSpec((1, 1, N), lambda sb: (sb, 0, 0)),
            _full(W_Q), _full(W_K), _full(W_V),
        ],
        out_specs=pl.BlockSpec((1, N, D), lambda sb: (sb, 0, 0)),
        out_shape=jax.ShapeDtypeStruct((TB, N, D), F32),
    )
    cross = att(f_all, fk_all, m_all, mk_T, W_Q, W_K, W_V)

    wn1_f = W_n1[0:D]
    wn1_agg = W_n1[D:2 * D]
    wn1_cross = W_n1[2 * D:3 * D]
    wn1_of = W_n1[3 * D:4 * D]
    node_weights = [row(g_nln1), row(b_nln1), wn1_f, wn1_agg, wn1_cross,
                    wn1_of, row(b_n1), row(g_nln2), row(b_nln2), W_n2,
                    row(b_n2), row(g_nln3), row(b_nln3)]
    node = pl.pallas_call(
        _node_body,
        grid=(TB,),
        in_specs=[
            pl.BlockSpec((1, N, 8), lambda sb: (sb, 0, 0)),
            pl.BlockSpec((1, N, D), lambda sb: (sb, 0, 0)),
            pl.BlockSpec((1, N, D), lambda sb: (sb, 0, 0)),
            pl.BlockSpec((1, N, 1), lambda sb: (sb, 0, 0)),
            pl.BlockSpec((1, N, D), lambda sb: (sb, 0, 0)),
            pl.BlockSpec((1, N, D), lambda sb: (sb, 0, 0)),
            pl.BlockSpec((1, N, 8), lambda sb: (sb, 0, 0)),
        ] + [_full(w) for w in node_weights],
        out_specs=[
            pl.BlockSpec((1, N, 8), lambda sb: (sb, 0, 0)),
            pl.BlockSpec((1, N, D), lambda sb: (sb, 0, 0)),
        ],
        out_shape=[
            jax.ShapeDtypeStruct((TB, N, 8), F32),
            jax.ShapeDtypeStruct((TB, N, D), F32),
        ],
    )
    c_out, f_out = node(c8, f_all, of_all, m_all, cross, msum, aux,
                        *node_weights)

    c_new = c_out[:, :, 0:3]
    return (c_new[:B], f_out[:B], c_new[B:], f_out[B:])
